# baseline (speedup 1.0000x reference)
import functools

import jax
import jax.numpy as jnp
import numpy as np
from jax.experimental import pallas as pl
from jax.experimental.pallas import tpu as pltpu


_ROT1 = (13, 15, 26, 6)
_ROT2 = (17, 29, 16, 24)


def _rotl(x, d):
    return (x << jnp.uint32(d)) | (x >> jnp.uint32(32 - d))


def _rounds(x0, x1, rots):
    for r in rots:
        x0 = x0 + x1
        x1 = _rotl(x1, r)
        x1 = x0 ^ x1
    return x0, x1


def _threefry_bits(i, k1, k2):
    ks0 = jnp.uint32(k1)
    ks1 = jnp.uint32(k2)
    ks2 = jnp.uint32(np.uint32(k1) ^ np.uint32(k2) ^ np.uint32(0x1BD11BDA))
    x0 = jnp.zeros_like(i) + ks0
    x1 = i + ks1
    x0, x1 = _rounds(x0, x1, _ROT1)
    x0, x1 = x0 + ks1, x1 + (ks2 + jnp.uint32(1))
    x0, x1 = _rounds(x0, x1, _ROT2)
    x0, x1 = x0 + ks2, x1 + (ks0 + jnp.uint32(2))
    x0, x1 = _rounds(x0, x1, _ROT1)
    x0, x1 = x0 + ks0, x1 + (ks1 + jnp.uint32(3))
    x0, x1 = _rounds(x0, x1, _ROT2)
    x0, x1 = x0 + ks1, x1 + (ks2 + jnp.uint32(4))
    x0, x1 = _rounds(x0, x1, _ROT1)
    x0, x1 = x0 + ks2, x1 + (ks0 + jnp.uint32(5))
    return x0 ^ x1


def _sample_kernel(x_ref, val_ref, idx_ref, *, ncols, block_cols):
    g = pl.program_id(0)
    j = pl.program_id(1)
    rows, cols = x_ref.shape

    col = jax.lax.broadcasted_iota(jnp.int32, (rows, cols), 1) + j * block_cols
    row = jax.lax.broadcasted_iota(jnp.int32, (rows, cols), 0) + g * rows
    flat = row.astype(jnp.uint32) * jnp.uint32(ncols) + col.astype(jnp.uint32)

    bits = _threefry_bits(flat, 0, 42)

    fbits = (bits >> jnp.uint32(9)) | jnp.uint32(0x3F800000)
    floats = jax.lax.bitcast_convert_type(fbits, jnp.float32) - jnp.float32(1.0)
    tiny = np.float32(np.finfo(np.float32).tiny)
    span = np.float32(np.float32(1.0) - tiny)
    u = jnp.maximum(tiny, floats * span + tiny)

    gumbel = -jnp.log(-jnp.log(u))
    vals = gumbel + x_ref[...]
    valid = col < ncols
    vals = jnp.where(valid, vals, -jnp.inf)

    bmax = jnp.max(vals, axis=1, keepdims=True)
    cand = jnp.where(vals == bmax, col, jnp.int32(np.iinfo(np.int32).max))
    barg = jnp.min(cand, axis=1, keepdims=True)

    @pl.when(j == 0)
    def _():
        val_ref[...] = bmax
        idx_ref[...] = barg

    @pl.when(j != 0)
    def _():
        upd = bmax > val_ref[...]
        idx_ref[...] = jnp.where(upd, barg, idx_ref[...])
        val_ref[...] = jnp.where(upd, bmax, val_ref[...])


@jax.jit
def kernel(log_p):
    rows, ncols = log_p.shape
    block_cols = 8192
    block_rows = 8
    gcols = pl.cdiv(ncols, block_cols)
    grows = pl.cdiv(rows, block_rows)
    _, idx = pl.pallas_call(
        functools.partial(_sample_kernel, ncols=ncols, block_cols=block_cols),
        grid=(grows, gcols),
        in_specs=[pl.BlockSpec((block_rows, block_cols), lambda g, j: (g, j))],
        out_specs=[
            pl.BlockSpec((block_rows, 1), lambda g, j: (g, 0)),
            pl.BlockSpec((block_rows, 1), lambda g, j: (g, 0)),
        ],
        out_shape=[
            jax.ShapeDtypeStruct((rows, 1), jnp.float32),
            jax.ShapeDtypeStruct((rows, 1), jnp.int32),
        ],
        compiler_params=pltpu.CompilerParams(
            dimension_semantics=("parallel", "arbitrary"),
        ),
    )(log_p)
    return idx[:, 0].astype(jnp.int64)


# speedup vs baseline: 1.6187x; 1.6187x over previous
import functools

import jax
import jax.numpy as jnp
import numpy as np
from jax.experimental import pallas as pl
from jax.experimental.pallas import tpu as pltpu


_ROT1 = (13, 15, 26, 6)
_ROT2 = (17, 29, 16, 24)


def _rotl(x, d):
    return (x << jnp.uint32(d)) | (x >> jnp.uint32(32 - d))


def _rounds(x0, x1, rots):
    for r in rots:
        x0 = x0 + x1
        x1 = _rotl(x1, r)
        x1 = x0 ^ x1
    return x0, x1


def _threefry_bits(i, k1, k2):
    ks0 = jnp.uint32(k1)
    ks1 = jnp.uint32(k2)
    ks2 = jnp.uint32(np.uint32(k1) ^ np.uint32(k2) ^ np.uint32(0x1BD11BDA))
    x0 = jnp.zeros_like(i) + ks0
    x1 = i + ks1
    x0, x1 = _rounds(x0, x1, _ROT1)
    x0, x1 = x0 + ks1, x1 + (ks2 + jnp.uint32(1))
    x0, x1 = _rounds(x0, x1, _ROT2)
    x0, x1 = x0 + ks2, x1 + (ks0 + jnp.uint32(2))
    x0, x1 = _rounds(x0, x1, _ROT1)
    x0, x1 = x0 + ks0, x1 + (ks1 + jnp.uint32(3))
    x0, x1 = _rounds(x0, x1, _ROT2)
    x0, x1 = x0 + ks1, x1 + (ks2 + jnp.uint32(4))
    x0, x1 = _rounds(x0, x1, _ROT1)
    x0, x1 = x0 + ks2, x1 + (ks0 + jnp.uint32(5))
    return x0 ^ x1


_NEG_LN2 = np.float32(-np.log(2.0))


def _gumbel_from_bits(bits):
    fbits = (bits >> jnp.uint32(9)) | jnp.uint32(0x3F800000)
    floats = jax.lax.bitcast_convert_type(fbits, jnp.float32) - jnp.float32(1.0)
    tiny = np.float32(np.finfo(np.float32).tiny)
    u = floats + tiny
    t = jnp.log2(u) * _NEG_LN2
    return jnp.log2(t) * _NEG_LN2


def _sample_kernel(x_ref, idx_ref, acc_val, acc_blk, *, ncols, block_cols):
    j = pl.program_id(0)
    nblocks = pl.num_programs(0)
    rows, cols = x_ref.shape

    @pl.when(j == 0)
    def _():
        acc_val[...] = jnp.full((rows, cols), -jnp.inf, jnp.float32)
        acc_blk[...] = jnp.zeros((rows, cols), jnp.int32)

    col = jax.lax.broadcasted_iota(jnp.int32, (rows, cols), 1) + j * block_cols
    row = jax.lax.broadcasted_iota(jnp.int32, (rows, cols), 0)
    flat = row.astype(jnp.uint32) * jnp.uint32(ncols) + col.astype(jnp.uint32)

    gumbel = _gumbel_from_bits(_threefry_bits(flat, 0, 42))
    vals = gumbel + x_ref[...]
    vals = jnp.where(col < ncols, vals, -jnp.inf)

    upd = vals > acc_val[...]
    acc_blk[...] = jnp.where(upd, j, acc_blk[...])
    acc_val[...] = jnp.where(upd, vals, acc_val[...])

    @pl.when(j == nblocks - 1)
    def _():
        av = acc_val[...]
        m = jnp.max(av, axis=1, keepdims=True)
        pos = jax.lax.broadcasted_iota(jnp.int32, (rows, cols), 1)
        cand = jnp.where(
            av == m,
            acc_blk[...] * block_cols + pos,
            jnp.int32(np.iinfo(np.int32).max),
        )
        idx_ref[...] = jnp.min(cand, axis=1, keepdims=True)


@jax.jit
def kernel(log_p):
    rows, ncols = log_p.shape
    block_cols = 2048
    grid = pl.cdiv(ncols, block_cols)
    idx = pl.pallas_call(
        functools.partial(_sample_kernel, ncols=ncols, block_cols=block_cols),
        grid=(grid,),
        in_specs=[pl.BlockSpec((rows, block_cols), lambda j: (0, j))],
        out_specs=pl.BlockSpec((rows, 1), lambda j: (0, 0)),
        out_shape=jax.ShapeDtypeStruct((rows, 1), jnp.int32),
        scratch_shapes=[
            pltpu.VMEM((rows, block_cols), jnp.float32),
            pltpu.VMEM((rows, block_cols), jnp.int32),
        ],
        compiler_params=pltpu.CompilerParams(
            dimension_semantics=("arbitrary",),
        ),
    )(log_p)
    return idx[:, 0].astype(jnp.int64)


# speedup vs baseline: 1.6441x; 1.0157x over previous
import functools

import jax
import jax.numpy as jnp
import numpy as np
from jax.experimental import pallas as pl
from jax.experimental.pallas import tpu as pltpu


_ROT1 = (13, 15, 26, 6)
_ROT2 = (17, 29, 16, 24)


def _rotl(x, d):
    return (x << jnp.uint32(d)) | (x >> jnp.uint32(32 - d))


def _rounds(x0, x1, rots):
    for r in rots:
        x0 = x0 + x1
        x1 = _rotl(x1, r)
        x1 = x0 ^ x1
    return x0, x1


def _threefry_bits(i, k1, k2):
    ks0 = jnp.uint32(k1)
    ks1 = jnp.uint32(k2)
    ks2 = jnp.uint32(np.uint32(k1) ^ np.uint32(k2) ^ np.uint32(0x1BD11BDA))
    x0 = jnp.zeros_like(i) + ks0
    x1 = i + ks1
    x0, x1 = _rounds(x0, x1, _ROT1)
    x0, x1 = x0 + ks1, x1 + (ks2 + jnp.uint32(1))
    x0, x1 = _rounds(x0, x1, _ROT2)
    x0, x1 = x0 + ks2, x1 + (ks0 + jnp.uint32(2))
    x0, x1 = _rounds(x0, x1, _ROT1)
    x0, x1 = x0 + ks0, x1 + (ks1 + jnp.uint32(3))
    x0, x1 = _rounds(x0, x1, _ROT2)
    x0, x1 = x0 + ks1, x1 + (ks2 + jnp.uint32(4))
    x0, x1 = _rounds(x0, x1, _ROT1)
    x0, x1 = x0 + ks2, x1 + (ks0 + jnp.uint32(5))
    return x0 ^ x1


_NEG_LN2 = np.float32(-np.log(2.0))


def _gumbel_from_bits(bits):
    fbits = (bits >> jnp.uint32(9)) | jnp.uint32(0x3F800000)
    floats = jax.lax.bitcast_convert_type(fbits, jnp.float32) - jnp.float32(1.0)
    tiny = np.float32(np.finfo(np.float32).tiny)
    u = floats + tiny
    t = jnp.log2(u) * _NEG_LN2
    return jnp.log2(t) * _NEG_LN2


def _sample_kernel(x_ref, idx_ref, acc_val, acc_blk, *, ncols, block_cols):
    j = pl.program_id(0)
    nblocks = pl.num_programs(0)
    rows, cols = x_ref.shape

    @pl.when(j == 0)
    def _():
        acc_val[...] = jnp.full((rows, cols), -jnp.inf, jnp.float32)
        acc_blk[...] = jnp.zeros((rows, cols), jnp.int32)

    col = jax.lax.broadcasted_iota(jnp.int32, (rows, cols), 1) + j * block_cols
    row = jax.lax.broadcasted_iota(jnp.int32, (rows, cols), 0)
    flat = row.astype(jnp.uint32) * jnp.uint32(ncols) + col.astype(jnp.uint32)

    gumbel = _gumbel_from_bits(_threefry_bits(flat, 0, 42))
    vals = gumbel + x_ref[...]
    vals = jnp.where(col < ncols, vals, -jnp.inf)

    upd = vals > acc_val[...]
    acc_blk[...] = jnp.where(upd, j, acc_blk[...])
    acc_val[...] = jnp.where(upd, vals, acc_val[...])

    @pl.when(j == nblocks - 1)
    def _():
        av = acc_val[...]
        m = jnp.max(av, axis=1, keepdims=True)
        pos = jax.lax.broadcasted_iota(jnp.int32, (rows, cols), 1)
        cand = jnp.where(
            av == m,
            acc_blk[...] * block_cols + pos,
            jnp.int32(np.iinfo(np.int32).max),
        )
        idx_ref[...] = jnp.min(cand, axis=1, keepdims=True)


@jax.jit
def kernel(log_p):
    rows, ncols = log_p.shape
    block_cols = 4096
    grid = pl.cdiv(ncols, block_cols)
    idx = pl.pallas_call(
        functools.partial(_sample_kernel, ncols=ncols, block_cols=block_cols),
        grid=(grid,),
        in_specs=[pl.BlockSpec((rows, block_cols), lambda j: (0, j))],
        out_specs=pl.BlockSpec((rows, 1), lambda j: (0, 0)),
        out_shape=jax.ShapeDtypeStruct((rows, 1), jnp.int32),
        scratch_shapes=[
            pltpu.VMEM((rows, block_cols), jnp.float32),
            pltpu.VMEM((rows, block_cols), jnp.int32),
        ],
        compiler_params=pltpu.CompilerParams(
            dimension_semantics=("arbitrary",),
        ),
    )(log_p)
    return idx[:, 0].astype(jnp.int64)


# speedup vs baseline: 1.6465x; 1.0015x over previous
import functools

import jax
import jax.numpy as jnp
import numpy as np
from jax.experimental import pallas as pl
from jax.experimental.pallas import tpu as pltpu


_ROT1 = (13, 15, 26, 6)
_ROT2 = (17, 29, 16, 24)


def _rotl(x, d):
    return (x << jnp.uint32(d)) | (x >> jnp.uint32(32 - d))


def _rounds(x0, x1, rots):
    for r in rots:
        x0 = x0 + x1
        x1 = _rotl(x1, r)
        x1 = x0 ^ x1
    return x0, x1


def _threefry_bits(i, k1, k2):
    ks0 = jnp.uint32(k1)
    ks1 = jnp.uint32(k2)
    ks2 = jnp.uint32(np.uint32(k1) ^ np.uint32(k2) ^ np.uint32(0x1BD11BDA))
    x0 = jnp.zeros_like(i) + ks0
    x1 = i + ks1
    x0, x1 = _rounds(x0, x1, _ROT1)
    x0, x1 = x0 + ks1, x1 + (ks2 + jnp.uint32(1))
    x0, x1 = _rounds(x0, x1, _ROT2)
    x0, x1 = x0 + ks2, x1 + (ks0 + jnp.uint32(2))
    x0, x1 = _rounds(x0, x1, _ROT1)
    x0, x1 = x0 + ks0, x1 + (ks1 + jnp.uint32(3))
    x0, x1 = _rounds(x0, x1, _ROT2)
    x0, x1 = x0 + ks1, x1 + (ks2 + jnp.uint32(4))
    x0, x1 = _rounds(x0, x1, _ROT1)
    x0, x1 = x0 + ks2, x1 + (ks0 + jnp.uint32(5))
    return x0 ^ x1


_NEG_LN2 = np.float32(-np.log(2.0))


def _gumbel_from_bits(bits):
    fbits = (bits >> jnp.uint32(9)) | jnp.uint32(0x3F800000)
    floats = jax.lax.bitcast_convert_type(fbits, jnp.float32) - jnp.float32(1.0)
    tiny = np.float32(np.finfo(np.float32).tiny)
    u = floats + tiny
    t = jnp.log2(u) * _NEG_LN2
    return jnp.log2(t) * _NEG_LN2


def _sample_kernel(x_ref, idx_ref, acc_val, acc_blk, *, ncols, block_cols):
    j = pl.program_id(0)
    nblocks = pl.num_programs(0)
    rows, cols = x_ref.shape

    @pl.when(j == 0)
    def _():
        acc_val[...] = jnp.full((rows, cols), -jnp.inf, jnp.float32)
        acc_blk[...] = jnp.zeros((rows, cols), jnp.int32)

    col = jax.lax.broadcasted_iota(jnp.int32, (rows, cols), 1) + j * block_cols
    row = jax.lax.broadcasted_iota(jnp.int32, (rows, cols), 0)
    flat = row.astype(jnp.uint32) * jnp.uint32(ncols) + col.astype(jnp.uint32)

    gumbel = _gumbel_from_bits(_threefry_bits(flat, 0, 42))
    vals = gumbel + x_ref[...]
    vals = jnp.where(col < ncols, vals, -jnp.inf)

    upd = vals > acc_val[...]
    acc_blk[...] = jnp.where(upd, j, acc_blk[...])
    acc_val[...] = jnp.where(upd, vals, acc_val[...])

    @pl.when(j == nblocks - 1)
    def _():
        av = acc_val[...]
        m = jnp.max(av, axis=1, keepdims=True)
        pos = jax.lax.broadcasted_iota(jnp.int32, (rows, cols), 1)
        cand = jnp.where(
            av == m,
            acc_blk[...] * block_cols + pos,
            jnp.int32(np.iinfo(np.int32).max),
        )
        idx_ref[...] = jnp.min(cand, axis=1, keepdims=True)


@jax.jit
def kernel(log_p):
    rows, ncols = log_p.shape
    block_cols = 8192
    grid = pl.cdiv(ncols, block_cols)
    idx = pl.pallas_call(
        functools.partial(_sample_kernel, ncols=ncols, block_cols=block_cols),
        grid=(grid,),
        in_specs=[pl.BlockSpec((rows, block_cols), lambda j: (0, j))],
        out_specs=pl.BlockSpec((rows, 1), lambda j: (0, 0)),
        out_shape=jax.ShapeDtypeStruct((rows, 1), jnp.int32),
        scratch_shapes=[
            pltpu.VMEM((rows, block_cols), jnp.float32),
            pltpu.VMEM((rows, block_cols), jnp.int32),
        ],
        compiler_params=pltpu.CompilerParams(
            dimension_semantics=("arbitrary",),
        ),
    )(log_p)
    return idx[:, 0].astype(jnp.int64)
